# baseline (speedup 1.0000x reference)
import math

import jax
import jax.numpy as jnp
from jax.experimental import pallas as pl

B, N, T, IN_DIM, H, C, OUT_DIM = 16, 300, 12, 2, 32, 8, 1
NP = 304
R = B * NP
_F32 = jnp.float32


def _dot(a, b):
    return jnp.dot(a, b, preferred_element_type=_F32)


def _call(fn, out_shapes, *args):
    return pl.pallas_call(
        fn,
        out_shape=[jax.ShapeDtypeStruct(s, _F32) for s in out_shapes],
    )(*args)


def _k_proj(x_ref, bd_ref, te_ref, wq_ref, wk_ref, xf_ref, cq_ref, ck_ref):
    xf_ref[...] = _dot(x_ref[...], bd_ref[...])
    cq_ref[...] = _dot(te_ref[...], wq_ref[...])
    ck_ref[...] = _dot(te_ref[...], wk_ref[...])


def _k_gru(tx_ref, hp_ref, gm_ref, wih_ref, whh_ref, bih_ref, bhh_ref,
           lat_ref):
    hp2 = hp_ref[...].reshape(R, H)

    def c_body(c, lat):
        ci = jax.lax.broadcasted_iota(jnp.int32, (NP, C), 1)
        m = jnp.sum(jnp.where(ci == c, gm_ref[...], 0.0), axis=1,
                    keepdims=True)
        mb = jnp.broadcast_to(m[None], (B, NP, 1)).reshape(R, 1)
        xm = tx_ref[...].reshape(R, 2 * H) * mb
        hm = hp2 * mb
        gi = _dot(xm, wih_ref[c]) + bih_ref[c]
        gh = _dot(hm, whh_ref[c]) + bhh_ref[c]
        r = jax.nn.sigmoid(gi[:, :H] + gh[:, :H])
        z = jax.nn.sigmoid(gi[:, H:2 * H] + gh[:, H:2 * H])
        n = jnp.tanh(gi[:, 2 * H:] + r * gh[:, 2 * H:])
        return lat + mb * ((1.0 - z) * n + z * hm)

    lat = jax.lax.fori_loop(0, C, c_body, jnp.zeros((R, H), _F32))
    lat_ref[...] = lat.reshape(B, NP, H)[:, :N, :]


def _k_p0(src_ref, tgt_ref, p0_ref):
    p0_ref[...] = _dot(src_ref[...], tgt_ref[...])


def _k_scores(lat_ref, hp_ref, cq_ref, ck_ref, s_ref):
    def b_body(b, carry):
        q = _dot(lat_ref[b], cq_ref[b])
        k = _dot(hp_ref[b], ck_ref[b])
        s_ref[b] = jax.lax.dot_general(q, k, (((1,), (1,)), ((), ())),
                                       preferred_element_type=_F32)
        return carry

    jax.lax.fori_loop(0, B, b_body, 0)


def _k_gcn2(adj_ref, h_ref, w_ref, b_ref, o_ref):
    def b_body(b, carry):
        hb = h_ref[b]
        adj = adj_ref[...]
        h1 = _dot(adj, hb)
        h2 = _dot(adj, h1)
        o_ref[b] = _dot(jnp.concatenate([hb, h1, h2], 1), w_ref[...]) + b_ref[...]
        return carry

    jax.lax.fori_loop(0, B, b_body, 0)


def _k_gcn3(adj_ref, h_ref, w_ref, b_ref, o_ref):
    def b_body(b, carry):
        hb = h_ref[b]
        adj = adj_ref[b]
        h1 = _dot(adj, hb)
        h2 = _dot(adj, h1)
        o_ref[b] = _dot(jnp.concatenate([hb, h1, h2], 1), w_ref[...]) + b_ref[...]
        return carry

    jax.lax.fori_loop(0, B, b_body, 0)


def _k_head(ch_ref, w1_ref, b1_ref, w2_ref, b2_ref, o_ref):
    h1 = jnp.maximum(_dot(ch_ref[...], w1_ref[...]) + b1_ref[...], 0.0)
    o_ref[...] = _dot(h1, w2_ref[...]) + b2_ref[...]


def _pad_rows(a3):
    return jnp.zeros((B, NP, a3.shape[-1]), _F32).at[:, :N].set(a3)


def kernel(x, t_pos, params, geo_graph, gumbel_u):
    p = params
    total_t = jnp.where(t_pos[:, :, 0] <= 4, t_pos[:, :, 1],
                        47 + t_pos[:, :, 1])
    half = H // 2
    freqs = jnp.exp(-math.log(10000.0) * jnp.arange(half, dtype=_F32) / half)
    targs = total_t.reshape(-1)[:, None].astype(_F32) * freqs[None]
    t_emb = jnp.concatenate([jnp.cos(targs), jnp.sin(targs)],
                            axis=-1).reshape(B, T, H)

    g = -jnp.log(-jnp.log(gumbel_u))
    y_soft = jax.nn.softmax(p['soft_mat'] + g, axis=1)
    idx = jnp.argmax(y_soft, axis=1)
    y_hard = jax.nn.one_hot(idx, C, dtype=_F32)
    gmat = jax.lax.stop_gradient(y_hard - y_soft) + y_soft
    gmat_pad = jnp.zeros((NP, C), _F32).at[:N].set(gmat)

    xflat = x.reshape(B, N, T * IN_DIM)
    xrows = _pad_rows(xflat).reshape(R, T * IN_DIM)
    bd = jnp.zeros((T * IN_DIM, T * H), _F32)
    for tt in range(T):
        bd = bd.at[IN_DIM * tt:IN_DIM * (tt + 1), H * tt:H * (tt + 1)].set(p['fc_x_t']['W'])
    te2 = t_emb.reshape(B * T, H)
    xf_all, cq_all, ck_all = _call(
        _k_proj, [(R, T * H), (B * T, H * H), (B * T, H * H)],
        xrows, bd, te2, p['core_fc_q']['W'], p['core_fc_k']['W'])
    xf_all = xf_all.reshape(B, NP, T * H)[:, :N, :]
    cq_all = (cq_all + p['core_fc_q']['b']).reshape(B, T, H, H)
    ck_all = (ck_all + p['core_fc_k']['b']).reshape(B, T, H, H)

    wih = jnp.stack([q['Wih'].T for q in p['grus']])
    whh = jnp.stack([q['Whh'].T for q in p['grus']])
    bih = jnp.stack([q['bih'][None] for q in p['grus']])
    bhh = jnp.stack([q['bhh'][None] for q in p['grus']])

    cur_h = jnp.zeros((B, N, H), _F32)
    for tt in range(T):
        xf = xf_all[:, :, H * tt:H * (tt + 1)] + p['fc_x_t']['b']
        te = jnp.broadcast_to(t_emb[:, None, tt, :], (B, N, H))
        total_x_t = jnp.concatenate([xf, te], axis=-1)
        h_prev = cur_h

        (latent,) = _call(
            _k_gru, [(B, N, H)],
            _pad_rows(total_x_t), _pad_rows(h_prev), gmat_pad,
            wih, whh, bih, bhh)

        mean = latent.mean(axis=(0, 1))
        var = ((latent - mean) ** 2).mean(axis=(0, 1))
        latent = (latent - mean) / jnp.sqrt(var + 1e-5) \
            * p['bn_gamma'][tt] + p['bn_beta'][tt]

        (p0,) = _call(_k_p0, [(N, N)], p['prior_src'][tt], p['prior_tgt'][tt])
        prior_dist = jax.nn.softmax(jax.nn.relu(p0), axis=1)

        (unbias,) = _call(_k_gcn2, [(B, N, H)],
                          prior_dist, latent,
                          p['backdoor'][tt]['W'], p['backdoor'][tt]['b'][None])
        (h_aug_geo,) = _call(_k_gcn2, [(B, N, H)],
                             geo_graph, h_prev,
                             p['prior_geo']['W'], p['prior_geo']['b'][None])

        (s_raw,) = _call(_k_scores, [(B, N, N)],
                         latent, h_prev, cq_all[:, tt], ck_all[:, tt])
        cmap = jax.nn.softmax(s_raw / math.sqrt(H), axis=-1)

        (h_aug_causal,) = _call(_k_gcn3, [(B, N, H)],
                                cmap, h_prev,
                                p['causal'][tt]['W'], p['causal'][tt]['b'][None])
        cur_h = h_aug_geo + h_aug_causal + unbias

    (out,) = _call(_k_head, [(B * N, OUT_DIM)],
                   cur_h.reshape(B * N, H), p['fcs_1']['W'],
                   p['fcs_1']['b'][None], p['fcs_2']['W'], p['fcs_2']['b'][None])
    return out.reshape(B, N, OUT_DIM)[:, :, None, :]
